# baseline (speedup 1.0000x reference)
import functools

import jax
import jax.numpy as jnp
from jax.experimental import pallas as pl
from jax.experimental.pallas import tpu as pltpu


def _pipe_body(x_hbm, w_ref, b_ref, o_hbm, x_buf, o_buf, in_sem, out_sem,
               *, n_steps):
    def dma_in(slot, step):
        pltpu.make_async_copy(
            x_hbm.at[step], x_buf.at[slot], in_sem.at[slot]).start()

    def wait_in(slot):
        pltpu.make_async_copy(
            x_hbm.at[0], x_buf.at[slot], in_sem.at[slot]).wait()

    def dma_out(slot, step):
        pltpu.make_async_copy(
            o_buf.at[slot], o_hbm.at[step], out_sem.at[slot]).start()

    def wait_out(slot):
        pltpu.make_async_copy(
            o_buf.at[slot], o_hbm.at[0], out_sem.at[slot]).wait()

    w = w_ref[...]
    b = b_ref[...]

    dma_in(0, 0)

    def body(step, _):
        cur = jax.lax.rem(step, 2)
        nxt = jax.lax.rem(step + 1, 2)

        @pl.when(step + 1 < n_steps)
        def _():
            dma_in(nxt, step + 1)

        wait_in(cur)

        @pl.when(step >= 2)
        def _():
            wait_out(cur)

        y = jax.lax.dot_general(
            w, x_buf[cur], (((1,), (0,)), ((), ())),
            preferred_element_type=jnp.float32)
        o_buf[cur] = y + b

        dma_out(cur, step)
        return ()

    jax.lax.fori_loop(0, n_steps, body, ())
    wait_out(jax.lax.rem(n_steps - 2, 2))
    wait_out(jax.lax.rem(n_steps - 1, 2))


@jax.jit
def _mult_alpha(x_nchw, weight, bias, alpha):
    N, Cin, H, W = x_nchw.shape
    Cout = weight.shape[0]
    HW = H * W
    dtype = x_nchw.dtype

    alpha = jnp.asarray(alpha, jnp.float32)
    w2 = (weight.reshape(Cout, Cin).astype(jnp.float32) * alpha)
    b2 = (bias.astype(jnp.float32) * alpha).reshape(Cout, 1)

    x3 = x_nchw.reshape(N, Cin, HW)

    body = functools.partial(_pipe_body, n_steps=N)

    out3 = pl.pallas_call(
        body,
        out_shape=jax.ShapeDtypeStruct((N, Cout, HW), dtype),
        in_specs=[
            pl.BlockSpec(memory_space=pl.ANY),
            pl.BlockSpec(memory_space=pltpu.VMEM),
            pl.BlockSpec(memory_space=pltpu.VMEM),
        ],
        out_specs=pl.BlockSpec(memory_space=pl.ANY),
        scratch_shapes=[
            pltpu.VMEM((2, Cin, HW), dtype),
            pltpu.VMEM((2, Cout, HW), jnp.float32),
            pltpu.SemaphoreType.DMA((2,)),
            pltpu.SemaphoreType.DMA((2,)),
        ],
        compiler_params=pltpu.CompilerParams(
            vmem_limit_bytes=48 * 1024 * 1024,
        ),
    )(x3, w2, b2)

    return out3.reshape(N, Cout, H, W)


def kernel(x_nchw, weight, bias, alpha):
    return _mult_alpha(x_nchw, weight, bias, alpha)


# speedup vs baseline: 1.0017x; 1.0017x over previous
import functools

import jax
import jax.numpy as jnp
from jax.experimental import pallas as pl
from jax.experimental.pallas import tpu as pltpu


def _pipe_body(alpha_ref, x_hbm, w_ref, b_ref, o_hbm, x_buf, o_buf,
               in_sem, out_sem, *, n_steps):
    def dma_in(slot, step):
        pltpu.make_async_copy(
            x_hbm.at[step], x_buf.at[slot], in_sem.at[slot]).start()

    def wait_in(slot):
        pltpu.make_async_copy(
            x_hbm.at[0], x_buf.at[slot], in_sem.at[slot]).wait()

    def dma_out(slot, step):
        pltpu.make_async_copy(
            o_buf.at[slot], o_hbm.at[step], out_sem.at[slot]).start()

    def wait_out(slot):
        pltpu.make_async_copy(
            o_buf.at[slot], o_hbm.at[0], out_sem.at[slot]).wait()

    alpha = alpha_ref[0]
    w = w_ref[...] * alpha
    b = b_ref[...] * alpha

    dma_in(0, 0)

    def body(step, _):
        cur = jax.lax.rem(step, 2)
        nxt = jax.lax.rem(step + 1, 2)

        @pl.when(step + 1 < n_steps)
        def _():
            dma_in(nxt, step + 1)

        wait_in(cur)

        @pl.when(step >= 2)
        def _():
            wait_out(cur)

        y = jax.lax.dot_general(
            w, x_buf[cur], (((1,), (0,)), ((), ())),
            preferred_element_type=jnp.float32)
        o_buf[cur] = y + b

        dma_out(cur, step)
        return ()

    jax.lax.fori_loop(0, n_steps, body, ())
    wait_out(jax.lax.rem(n_steps - 2, 2))
    wait_out(jax.lax.rem(n_steps - 1, 2))


@jax.jit
def _mult_alpha(x_nchw, weight, bias, alpha):
    N, Cin, H, W = x_nchw.shape
    Cout = weight.shape[0]
    HW = H * W
    dtype = x_nchw.dtype

    alpha1 = jnp.asarray(alpha, jnp.float32).reshape(1)
    w0 = weight.reshape(Cout, Cin)
    b0 = bias.reshape(Cout, 1)

    x3 = x_nchw.reshape(N, Cin, HW)

    body = functools.partial(_pipe_body, n_steps=N)

    out3 = pl.pallas_call(
        body,
        out_shape=jax.ShapeDtypeStruct((N, Cout, HW), dtype),
        in_specs=[
            pl.BlockSpec(memory_space=pltpu.SMEM),
            pl.BlockSpec(memory_space=pl.ANY),
            pl.BlockSpec(memory_space=pltpu.VMEM),
            pl.BlockSpec(memory_space=pltpu.VMEM),
        ],
        out_specs=pl.BlockSpec(memory_space=pl.ANY),
        scratch_shapes=[
            pltpu.VMEM((2, Cin, HW), dtype),
            pltpu.VMEM((2, Cout, HW), jnp.float32),
            pltpu.SemaphoreType.DMA((2,)),
            pltpu.SemaphoreType.DMA((2,)),
        ],
        compiler_params=pltpu.CompilerParams(
            vmem_limit_bytes=48 * 1024 * 1024,
        ),
    )(alpha1, x3, w0, b0)

    return out3.reshape(N, Cout, H, W)


def kernel(x_nchw, weight, bias, alpha):
    return _mult_alpha(x_nchw, weight, bias, alpha)
